# baseline (speedup 1.0000x reference)
import functools

import jax
import jax.numpy as jnp
from jax import lax
from jax.experimental import pallas as pl
from jax.experimental.pallas import tpu as pltpu
from jax.experimental.pallas import tpu_sc as plsc

D = 512
S = 50
C = 32
NBUF = 3
BB = 4


def _sc_gather(table, idx):
    N = idx.shape[0]
    info = plsc.get_sparse_core_info()
    NC, NS = info.num_cores, info.num_subcores
    NW = NC * NS
    n_w = N // NW
    n_chunks = n_w // C
    mesh = plsc.VectorSubcoreMesh(core_axis_name="c", subcore_axis_name="s")

    @functools.partial(
        pl.kernel,
        mesh=mesh,
        out_type=jax.ShapeDtypeStruct((N, D), jnp.float32),
        scratch_types=[
            pltpu.VMEM((n_w,), jnp.int32),
            pltpu.VMEM((NBUF, C, D), jnp.float32),
            pltpu.SemaphoreType.DMA((NBUF,)),
            pltpu.SemaphoreType.DMA((NBUF,)),
        ],
    )
    def k(tab_hbm, idx_hbm, out_hbm, idx_all, rows_v, sem_g, sem_o):
        wid = lax.axis_index("s") * NC + lax.axis_index("c")
        wbase = wid * n_w
        pltpu.sync_copy(idx_hbm.at[pl.ds(wbase, n_w)], idx_all)

        def gather_copy(g, b):
            ibase = pl.multiple_of(g * C, C)
            return pltpu.make_async_copy(
                tab_hbm.at[idx_all.at[pl.ds(ibase, C)]],
                rows_v.at[b], sem_g.at[b])

        def out_copy(g, b):
            base = pl.multiple_of(wbase + g * C, C)
            return pltpu.make_async_copy(
                rows_v.at[b], out_hbm.at[pl.ds(base, C)], sem_o.at[b])

        def issue_in(g):
            gather_copy(g, lax.rem(g, NBUF)).start()

        for g0 in range(NBUF - 1):
            issue_in(g0)

        def body(g, carry):
            b = lax.rem(g, NBUF)
            gather_copy(g, b).wait()
            out_copy(g, b).start()

            g2 = g + NBUF - 1
            b2 = lax.rem(g2, NBUF)

            @pl.when(jnp.logical_and(g >= 1, g2 < n_chunks))
            def _():
                out_copy(g - 1, b2).wait()

            @pl.when(g2 < n_chunks)
            def _():
                issue_in(g2)

            return carry

        lax.fori_loop(0, n_chunks, body, 0)

        for gd in range(n_chunks - NBUF, n_chunks):
            out_copy(gd, gd % NBUF).wait()

    return k(table, idx)


def _tc_add_kernel(x_ref, emb_ref, o_ref):
    for kk in range(BB):
        o_ref[kk] = x_ref[kk] + emb_ref[pl.ds(S * kk, S), :]


def _tc_add(x, emb):
    B = x.shape[0]
    grid = (B // BB,)
    return pl.pallas_call(
        _tc_add_kernel,
        grid=grid,
        in_specs=[
            pl.BlockSpec((BB, S, D), lambda i: (i, 0, 0)),
            pl.BlockSpec((BB * S, D), lambda i: (i, 0)),
        ],
        out_specs=pl.BlockSpec((BB, S, D), lambda i: (i, 0, 0)),
        out_shape=jax.ShapeDtypeStruct((B, S, D), jnp.float32),
    )(x, emb)


def kernel(x, instruction_types, type_emb_weight):
    idx = instruction_types.reshape(-1).astype(jnp.int32)
    emb = _sc_gather(type_emb_weight, idx)
    return _tc_add(x, emb)


# speedup vs baseline: 1.0003x; 1.0003x over previous
import functools

import jax
import jax.numpy as jnp
from jax import lax
from jax.experimental import pallas as pl
from jax.experimental.pallas import tpu as pltpu
from jax.experimental.pallas import tpu_sc as plsc

D = 512
S = 50
C = 64
NQ = 4
NBUF = 3
BB = 4


def _sc_gather(table, idx):
    N = idx.shape[0]
    info = plsc.get_sparse_core_info()
    NC, NS = info.num_cores, info.num_subcores
    NW = NC * NS
    n_w = N // NW
    n_chunks = n_w // C
    mesh = plsc.VectorSubcoreMesh(core_axis_name="c", subcore_axis_name="s")

    @functools.partial(
        pl.kernel,
        mesh=mesh,
        out_type=jax.ShapeDtypeStruct((N, D), jnp.float32),
        scratch_types=[
            pltpu.VMEM((n_w,), jnp.int32),
            pltpu.VMEM((NBUF, C, D), jnp.float32),
            pltpu.SemaphoreType.DMA((NBUF, NQ)),
            pltpu.SemaphoreType.DMA((NBUF,)),
        ],
    )
    def k(tab_hbm, idx_hbm, out_hbm, idx_all, rows_v, sem_g, sem_o):
        wid = lax.axis_index("s") * NC + lax.axis_index("c")
        wbase = wid * n_w
        pltpu.sync_copy(idx_hbm.at[pl.ds(wbase, n_w)], idx_all)
        CQ = C // NQ

        def gather_copy(g, b, q):
            ibase = pl.multiple_of(g * C + q * CQ, CQ)
            return pltpu.make_async_copy(
                tab_hbm.at[idx_all.at[pl.ds(ibase, CQ)]],
                rows_v.at[b, pl.ds(q * CQ, CQ)], sem_g.at[b, q])

        def out_copy(g, b):
            base = pl.multiple_of(wbase + g * C, C)
            return pltpu.make_async_copy(
                rows_v.at[b], out_hbm.at[pl.ds(base, C)], sem_o.at[b])

        def issue_in(g):
            b = lax.rem(g, NBUF)
            for q in range(NQ):
                gather_copy(g, b, q).start()

        for g0 in range(NBUF - 1):
            issue_in(g0)

        def body(g, carry):
            b = lax.rem(g, NBUF)
            for q in range(NQ):
                gather_copy(g, b, q).wait()
            out_copy(g, b).start()

            g2 = g + NBUF - 1
            b2 = lax.rem(g2, NBUF)

            @pl.when(jnp.logical_and(g >= 1, g2 < n_chunks))
            def _():
                out_copy(g - 1, b2).wait()

            @pl.when(g2 < n_chunks)
            def _():
                issue_in(g2)

            return carry

        lax.fori_loop(0, n_chunks, body, 0)

        for gd in range(n_chunks - NBUF, n_chunks):
            out_copy(gd, gd % NBUF).wait()

    return k(table, idx)


def _tc_add_kernel(x_ref, emb_ref, o_ref):
    for kk in range(BB):
        o_ref[kk] = x_ref[kk] + emb_ref[pl.ds(S * kk, S), :]


def _tc_add(x, emb):
    B = x.shape[0]
    grid = (B // BB,)
    return pl.pallas_call(
        _tc_add_kernel,
        grid=grid,
        in_specs=[
            pl.BlockSpec((BB, S, D), lambda i: (i, 0, 0)),
            pl.BlockSpec((BB * S, D), lambda i: (i, 0)),
        ],
        out_specs=pl.BlockSpec((BB, S, D), lambda i: (i, 0, 0)),
        out_shape=jax.ShapeDtypeStruct((B, S, D), jnp.float32),
    )(x, emb)


def kernel(x, instruction_types, type_emb_weight):
    idx = instruction_types.reshape(-1).astype(jnp.int32)
    emb = _sc_gather(type_emb_weight, idx)
    return _tc_add(x, emb)
